# baseline (speedup 1.0000x reference)
import functools

import jax
import jax.numpy as jnp
import numpy as np
from jax import lax
from jax.experimental import pallas as pl
from jax.experimental.pallas import tpu as pltpu
from jax.experimental.pallas import tpu_sc as plsc

_N = 2097152
_NIDX = 262144
_EPS = 1e-07
_ONE_PLUS_EPS = float(np.float32(1.0) + np.float32(_EPS))

_NC = 2
_NS = 16
_HALF = _N // _NC
_DUMMY = _HALF
_IDXC = _NIDX // _NS
_SLICE = _HALF // _NS

_mesh = plsc.VectorSubcoreMesh(core_axis_name="c", subcore_axis_name="s")


@functools.partial(
    pl.kernel,
    mesh=_mesh,
    out_type=jax.ShapeDtypeStruct((_N,), jnp.float32),
    scratch_types=[
        pltpu.VMEM((_IDXC,), jnp.int32),
        pltpu.VMEM((_IDXC,), jnp.float32),
        pltpu.VMEM((_IDXC,), jnp.float32),
        pltpu.VMEM_SHARED((_HALF + 8,), jnp.float32),
        pltpu.SemaphoreType.DMA,
        pltpu.SemaphoreType.DMA,
    ],
)
def _voxel_scatter(idx_hbm, out_hbm, idx_v, vals_v, eps_v, spmem, sem_a, sem_b):
    c = lax.axis_index("c")
    s = lax.axis_index("s")
    base = c * _HALF

    idx_load = pltpu.make_async_copy(
        idx_hbm.at[pl.ds(s * _IDXC, _IDXC)], idx_v, sem_a
    )
    idx_load.start()

    eps16 = jnp.full((16,), _EPS, jnp.float32)
    one16 = jnp.full((16,), _ONE_PLUS_EPS, jnp.float32)

    @pl.loop(0, _IDXC // 16, unroll=8)
    def _fill_eps(i):
        eps_v[pl.ds(i * 16, 16)] = eps16

    @pl.loop(0, _IDXC // 16, unroll=8)
    def _fill_vals(i):
        vals_v[pl.ds(i * 16, 16)] = one16

    inits = []
    for k in range(_SLICE // _IDXC):
        cp = pltpu.make_async_copy(
            eps_v, spmem.at[pl.ds(s * _SLICE + k * _IDXC, _IDXC)], sem_b
        )
        cp.start()
        inits.append(cp)

    idx_load.wait()
    dummy16 = jnp.full((16,), _DUMMY, jnp.int32)

    with jax.named_scope("rebase"):
        @pl.loop(0, _IDXC // 16, unroll=4)
        def _rebase(i):
            v = idx_v[pl.ds(i * 16, 16)]
            local = v - base
            m = (v >= base) & (local < _HALF)
            idx_v[pl.ds(i * 16, 16)] = jnp.where(m, local, dummy16)

    with jax.named_scope("init_wait"):
        for cp in inits:
            cp.wait()
        plsc.subcore_barrier()

    with jax.named_scope("scatter"):
        pltpu.async_copy(vals_v, spmem.at[idx_v], sem_a).wait()
        plsc.subcore_barrier()

    with jax.named_scope("writeback"):
        pltpu.sync_copy(
            spmem.at[pl.ds(s * _SLICE, _SLICE)],
            out_hbm.at[pl.ds(base + s * _SLICE, _SLICE)],
        )


def kernel(voxel, fish_cell_indices, cell_indices):
    del voxel, cell_indices
    return _voxel_scatter(fish_cell_indices)


# speedup vs baseline: 1.0001x; 1.0001x over previous
import functools

import jax
import jax.numpy as jnp
import numpy as np
from jax import lax
from jax.experimental import pallas as pl
from jax.experimental.pallas import tpu as pltpu
from jax.experimental.pallas import tpu_sc as plsc

_N = 2097152
_NIDX = 262144
_EPS = 1e-07
_ONE_PLUS_EPS = float(np.float32(1.0) + np.float32(_EPS))

_NC = 2
_NS = 16
_HALF = _N // _NC
_DUMMY = _HALF
_IDXC = _NIDX // _NS
_SLICE = _HALF // _NS

_mesh = plsc.VectorSubcoreMesh(core_axis_name="c", subcore_axis_name="s")


@functools.partial(
    pl.kernel,
    mesh=_mesh,
    out_type=jax.ShapeDtypeStruct((_N,), jnp.float32),
    scratch_types=[
        pltpu.VMEM((_IDXC,), jnp.int32),
        pltpu.VMEM((_IDXC,), jnp.float32),
        pltpu.VMEM((_IDXC,), jnp.float32),
        pltpu.VMEM_SHARED((_HALF + 8,), jnp.float32),
        pltpu.SemaphoreType.DMA,
        pltpu.SemaphoreType.DMA,
    ],
)
def _voxel_scatter(idx_hbm, out_hbm, idx_v, vals_v, eps_v, spmem, sem_a, sem_b):
    c = lax.axis_index("c")
    s = lax.axis_index("s")
    base = c * _HALF

    idx_load = pltpu.make_async_copy(
        idx_hbm.at[pl.ds(s * _IDXC, _IDXC)], idx_v, sem_a
    )
    idx_load.start()

    eps16 = jnp.full((16,), _EPS, jnp.float32)
    one16 = jnp.full((16,), _ONE_PLUS_EPS, jnp.float32)

    @pl.loop(0, _IDXC // 16, unroll=8)
    def _fill_eps(i):
        eps_v[pl.ds(i * 16, 16)] = eps16

    @pl.loop(0, _IDXC // 16, unroll=8)
    def _fill_vals(i):
        vals_v[pl.ds(i * 16, 16)] = one16

    inits = []
    for k in range(_SLICE // _IDXC):
        cp = pltpu.make_async_copy(
            eps_v, spmem.at[pl.ds(s * _SLICE + k * _IDXC, _IDXC)], sem_b
        )
        cp.start()
        inits.append(cp)

    idx_load.wait()
    dummy16 = jnp.full((16,), _DUMMY, jnp.int32)

    with jax.named_scope("rebase"):
        @pl.loop(0, _IDXC // 16, unroll=4)
        def _rebase(i):
            v = idx_v[pl.ds(i * 16, 16)]
            local = v - base
            m = (v >= base) & (local < _HALF)
            idx_v[pl.ds(i * 16, 16)] = jnp.where(m, local, dummy16)

    with jax.named_scope("init_wait"):
        for cp in inits:
            cp.wait()
        plsc.subcore_barrier()

    with jax.named_scope("scatter"):
        q = _IDXC // 4
        copies = []
        for j in range(4):
            cp = pltpu.make_async_copy(
                vals_v.at[pl.ds(j * q, q)],
                spmem.at[idx_v.at[pl.ds(j * q, q)]],
                sem_a,
            )
            cp.start()
            copies.append(cp)
        for cp in copies:
            cp.wait()
        plsc.subcore_barrier()

    with jax.named_scope("writeback"):
        pltpu.sync_copy(
            spmem.at[pl.ds(s * _SLICE, _SLICE)],
            out_hbm.at[pl.ds(base + s * _SLICE, _SLICE)],
        )


def kernel(voxel, fish_cell_indices, cell_indices):
    del voxel, cell_indices
    return _voxel_scatter(fish_cell_indices)


# speedup vs baseline: 1.0001x; 1.0000x over previous
import functools

import jax
import jax.numpy as jnp
import numpy as np
from jax import lax
from jax.experimental import pallas as pl
from jax.experimental.pallas import tpu as pltpu
from jax.experimental.pallas import tpu_sc as plsc

_N = 2097152
_NIDX = 262144
_EPS = 1e-07
_ONE_PLUS_EPS = float(np.float32(1.0) + np.float32(_EPS))

_NC = 2
_NS = 16
_HALF = _N // _NC
_DUMMY = _HALF
_IDXC = _NIDX // _NS
_SLICE = _HALF // _NS

_mesh = plsc.VectorSubcoreMesh(core_axis_name="c", subcore_axis_name="s")


@functools.partial(
    pl.kernel,
    mesh=_mesh,
    out_type=jax.ShapeDtypeStruct((_N,), jnp.float32),
    scratch_types=[
        pltpu.VMEM((_IDXC,), jnp.int32),
        pltpu.VMEM((_IDXC,), jnp.float32),
        pltpu.VMEM((_IDXC,), jnp.float32),
        pltpu.VMEM_SHARED((_HALF + 8,), jnp.float32),
        pltpu.SemaphoreType.DMA,
        pltpu.SemaphoreType.DMA,
    ],
)
def _voxel_scatter(idx_hbm, out_hbm, idx_v, vals_v, eps_v, spmem, sem_a, sem_b):
    c = lax.axis_index("c")
    s = lax.axis_index("s")
    base = c * _HALF

    idx_load = pltpu.make_async_copy(
        idx_hbm.at[pl.ds(s * _IDXC, _IDXC)], idx_v, sem_a
    )
    idx_load.start()

    eps16 = jnp.full((16,), _EPS, jnp.float32)
    one16 = jnp.full((16,), _ONE_PLUS_EPS, jnp.float32)

    @pl.loop(0, _IDXC // 16, unroll=8)
    def _fill_eps(i):
        eps_v[pl.ds(i * 16, 16)] = eps16

    @pl.loop(0, _IDXC // 16, unroll=8)
    def _fill_vals(i):
        vals_v[pl.ds(i * 16, 16)] = one16

    inits = []
    for k in range(_SLICE // _IDXC):
        cp = pltpu.make_async_copy(
            eps_v, spmem.at[pl.ds(s * _SLICE + k * _IDXC, _IDXC)], sem_b
        )
        cp.start()
        inits.append(cp)

    idx_load.wait()
    dummy16 = jnp.full((16,), _DUMMY, jnp.int32)

    with jax.named_scope("rebase"):
        @pl.loop(0, _IDXC // 16, unroll=4)
        def _rebase(i):
            v = idx_v[pl.ds(i * 16, 16)]
            local = v - base
            m = (v >= base) & (local < _HALF)
            idx_v[pl.ds(i * 16, 16)] = jnp.where(m, local, dummy16)

    with jax.named_scope("init_wait"):
        for cp in inits:
            cp.wait()
        plsc.subcore_barrier()

    with jax.named_scope("scatter"):
        pltpu.async_copy(vals_v, spmem.at[idx_v], sem_a).wait()
        plsc.subcore_barrier()

    with jax.named_scope("writeback"):
        pltpu.sync_copy(
            spmem.at[pl.ds(s * _SLICE, _SLICE)],
            out_hbm.at[pl.ds(base + s * _SLICE, _SLICE)],
        )


def kernel(voxel, fish_cell_indices, cell_indices):
    del voxel, cell_indices
    return _voxel_scatter(fish_cell_indices)


# speedup vs baseline: 3.1098x; 3.1094x over previous
import functools

import jax
import jax.numpy as jnp
import numpy as np
from jax import lax
from jax.experimental import pallas as pl
from jax.experimental.pallas import tpu as pltpu
from jax.experimental.pallas import tpu_sc as plsc

_N = 2097152
_NIDX = 262144
_EPS = 1e-07
_ONE_PLUS_EPS = float(np.float32(1.0) + np.float32(_EPS))

_NC = 2
_NS = 16
_HALF = _N // _NC
_DUMMY = _HALF
_IDXC = _NIDX // _NS
_SLICE = _HALF // _NS

_mesh = plsc.VectorSubcoreMesh(core_axis_name="c", subcore_axis_name="s")


@functools.partial(
    pl.kernel,
    mesh=_mesh,
    out_type=jax.ShapeDtypeStruct((_N,), jnp.float32),
    scratch_types=[
        pltpu.VMEM((_IDXC,), jnp.int32),
        pltpu.VMEM((_IDXC,), jnp.float32),
        pltpu.VMEM((_IDXC,), jnp.float32),
        pltpu.VMEM_SHARED((_HALF + 1024,), jnp.float32),
        pltpu.SemaphoreType.DMA,
        pltpu.SemaphoreType.DMA,
    ],
)
def _voxel_scatter(idx_hbm, out_hbm, idx_v, vals_v, eps_v, spmem, sem_a, sem_b):
    c = lax.axis_index("c")
    s = lax.axis_index("s")
    base = c * _HALF

    idx_load = pltpu.make_async_copy(
        idx_hbm.at[pl.ds(s * _IDXC, _IDXC)], idx_v, sem_a
    )
    idx_load.start()

    eps16 = jnp.full((16,), _EPS, jnp.float32)
    one16 = jnp.full((16,), _ONE_PLUS_EPS, jnp.float32)

    @pl.loop(0, _IDXC // 16, unroll=8)
    def _fill_eps(i):
        eps_v[pl.ds(i * 16, 16)] = eps16

    @pl.loop(0, _IDXC // 16, unroll=8)
    def _fill_vals(i):
        vals_v[pl.ds(i * 16, 16)] = one16

    inits = []
    for k in range(_SLICE // _IDXC):
        cp = pltpu.make_async_copy(
            eps_v, spmem.at[pl.ds(s * _SLICE + k * _IDXC, _IDXC)], sem_b
        )
        cp.start()
        inits.append(cp)

    idx_load.wait()
    dummy16 = _DUMMY + lax.iota(jnp.int32, 16)

    with jax.named_scope("rebase"):
        @pl.loop(0, _IDXC // 16, unroll=4)
        def _rebase(i):
            v = idx_v[pl.ds(i * 16, 16)]
            local = v - base
            m = (v >= base) & (local < _HALF)
            idx_v[pl.ds(i * 16, 16)] = jnp.where(
                m, local, dummy16 + (i & 63) * 16
            )

    with jax.named_scope("init_wait"):
        for cp in inits:
            cp.wait()
        plsc.subcore_barrier()

    with jax.named_scope("scatter"):
        pltpu.async_copy(vals_v, spmem.at[idx_v], sem_a).wait()
        plsc.subcore_barrier()

    with jax.named_scope("writeback"):
        pltpu.sync_copy(
            spmem.at[pl.ds(s * _SLICE, _SLICE)],
            out_hbm.at[pl.ds(base + s * _SLICE, _SLICE)],
        )


def kernel(voxel, fish_cell_indices, cell_indices):
    del voxel, cell_indices
    return _voxel_scatter(fish_cell_indices)
